# baseline (speedup 1.0000x reference)
import functools

import jax
import jax.numpy as jnp
from jax import lax
from jax.experimental import pallas as pl
from jax.experimental.pallas import tpu as pltpu
from jax.experimental.pallas import tpu_sc as plsc

B = 16384
D = 32
PK = 128 // D
VP = 1000000 // PK
NC, NS = 2, 16
NW = NC * NS
BPW = B // NW


@functools.cache
def _make_sc_gather():
    mesh = plsc.VectorSubcoreMesh(core_axis_name="c", subcore_axis_name="s")

    @functools.partial(
        pl.kernel,
        out_type=[jax.ShapeDtypeStruct((B, 128), jnp.float32)] * 4,
        mesh=mesh,
        scratch_types=[
            pltpu.VMEM((BPW,), jnp.int32),
            pltpu.VMEM((BPW,), jnp.int32),
            pltpu.VMEM((BPW // 2, 128), jnp.float32),
            pltpu.VMEM((BPW // 2, 128), jnp.float32),
            pltpu.SemaphoreType.DMA,
            pltpu.SemaphoreType.DMA,
            pltpu.SemaphoreType.DMA,
        ],
    )
    def sc_gather(uidx_hbm, iidx_hbm, gu_hbm, gi_hbm, mu_hbm, mi_hbm,
                  gu_out, gi_out, mu_out, mi_out,
                  ju_v, ji_v, rows_a, rows_b, sema, semb, semo):
        wid = lax.axis_index("s") * NC + lax.axis_index("c")
        base = wid * BPW
        pltpu.sync_copy(uidx_hbm.at[pl.ds(base, BPW)], ju_v)
        pltpu.sync_copy(iidx_hbm.at[pl.ds(base, BPW)], ji_v)

        def to_packed(k, _):
            ju_v[pl.ds(k * 16, 16)] = ju_v[pl.ds(k * 16, 16)] >> 2
            ji_v[pl.ds(k * 16, 16)] = ji_v[pl.ds(k * 16, 16)] >> 2
            return ()

        lax.fori_loop(0, BPW // 16, to_packed, (), unroll=4)

        CH = BPW // 2
        bufs = (rows_a, rows_b)
        sems = (sema, semb)
        rounds = []
        for tbl, jv, out in ((gu_hbm, ju_v, gu_out), (gi_hbm, ji_v, gi_out),
                             (mu_hbm, ju_v, mu_out), (mi_hbm, ji_v, mi_out)):
            for c in range(2):
                rounds.append((tbl, jv, out, c))
        n = len(rounds)
        gathers = [None] * n
        stores = [None] * n

        def finish(i):
            tbl, jv, out, c = rounds[i]
            gathers[i].wait()
            stores[i] = pltpu.async_copy(
                bufs[i % 2], out.at[pl.ds(base + c * CH, CH)], semo)

        for i, (tbl, jv, out, c) in enumerate(rounds):
            if i >= 2:
                stores[i - 2].wait()
            gathers[i] = pltpu.async_copy(
                tbl.at[jv.at[pl.ds(c * CH, CH)]], bufs[i % 2], sems[i % 2])
            if i >= 1:
                finish(i - 1)
        finish(n - 1)
        stores[n - 2].wait()
        stores[n - 1].wait()

    return sc_gather


BLK = 2048


def _mlp_body(uidx_ref, iidx_ref, gu_ref, gi_ref, mu_ref, mi_ref, w1_ref,
              b1_ref, w2_ref, b2_ref, w3_ref, b3_ref, wpg_ref, wph_ref,
              bp_ref, out_ref):
    uq = uidx_ref[...] & 3
    iq = iidx_ref[...] & 3

    def pick(raw, q):
        acc = jnp.zeros((BLK, D), jnp.float32)
        for c in range(PK):
            sel = (q == c).astype(jnp.float32)[:, None]
            acc = acc + raw[:, c * D:(c + 1) * D] * sel
        return acc

    gmf = pick(gu_ref[...], uq) * pick(gi_ref[...], iq)
    x = jnp.concatenate([pick(mu_ref[...], uq), pick(mi_ref[...], iq)],
                        axis=1)
    h = jnp.maximum(jnp.dot(x, w1_ref[...],
                            preferred_element_type=jnp.float32) + b1_ref[...],
                    0.0)
    h = jnp.maximum(jnp.dot(h, w2_ref[...],
                            preferred_element_type=jnp.float32) + b2_ref[...],
                    0.0)
    h = jnp.maximum(jnp.dot(h, w3_ref[...],
                            preferred_element_type=jnp.float32) + b3_ref[...],
                    0.0)
    logit = (jnp.dot(gmf, wpg_ref[...], preferred_element_type=jnp.float32)
             + jnp.dot(h, wph_ref[...], preferred_element_type=jnp.float32)
             + bp_ref[0, 0])
    out_ref[...] = jax.nn.sigmoid(logit)


def _run_mlp(uidx, iidx, gu, gi, mu, mi, W1, b1, W2, b2, W3, b3, Wpg, Wph,
             bp):
    grid = (B // BLK,)
    raw_spec = pl.BlockSpec((BLK, 128), lambda i: (i, 0))
    idx_spec = pl.BlockSpec((BLK,), lambda i: (i,))

    def whole(shape):
        return pl.BlockSpec(shape, lambda i: (0,) * len(shape))

    out = pl.pallas_call(
        _mlp_body,
        grid=grid,
        in_specs=[
            idx_spec, idx_spec,
            raw_spec, raw_spec, raw_spec, raw_spec,
            whole((64, 32)), whole((1, 32)),
            whole((32, 16)), whole((1, 16)),
            whole((16, 8)), whole((1, 8)),
            whole((32, 1)), whole((8, 1)), whole((1, 1)),
        ],
        out_specs=pl.BlockSpec((BLK, 1), lambda i: (i, 0)),
        out_shape=jax.ShapeDtypeStruct((B, 1), jnp.float32),
    )(uidx, iidx, gu, gi, mu, mi, W1, b1.reshape(1, 32), W2,
      b2.reshape(1, 16), W3, b3.reshape(1, 8), Wpg, Wph, bp.reshape(1, 1))
    return out.reshape(B)


def kernel(user_idx, item_idx, gmf_user, gmf_item, mlp_user, mlp_item,
           W1, b1, W2, b2, W3, b3, Wp, bp):
    uidx = user_idx.astype(jnp.int32)
    iidx = item_idx.astype(jnp.int32)
    gu_p = gmf_user.reshape(VP, 128)
    gi_p = gmf_item.reshape(VP, 128)
    mu_p = mlp_user.reshape(VP, 128)
    mi_p = mlp_item.reshape(VP, 128)
    gu, gi, mu, mi = _make_sc_gather()(uidx, iidx, gu_p, gi_p, mu_p, mi_p)
    Wpg = Wp[:D]
    Wph = Wp[D:]
    return _run_mlp(uidx, iidx, gu, gi, mu, mi, W1, b1, W2, b2, W3, b3,
                    Wpg, Wph, bp)
